# baseline (speedup 1.0000x reference)
import functools

import jax
import jax.numpy as jnp
from jax import lax
from jax.experimental import pallas as pl
from jax.experimental.pallas import tpu as pltpu
from jax.experimental.pallas import tpu_sc as plsc

N_TOK = 2048
C = 768
E = 8
D_FF = 2048
BLK = 128
G_ROWS = 2 * N_TOK + E * BLK
N_BLOCKS = G_ROWS // BLK
NC, NS = 2, 16
NW = NC * NS
TPW = N_TOK // NW
CHK = 32
N_CHK = TPW // CHK


@functools.lru_cache(maxsize=1)
def _sc_mesh():
  return plsc.VectorSubcoreMesh(
      core_axis_name="c", subcore_axis_name="s", num_cores=NC, num_subcores=NS)


def _cumsum_rows(a):
  n = a.shape[0]
  s = 1
  while s < n:
    shifted = jnp.concatenate(
        [jnp.zeros((s, a.shape[1]), a.dtype), a[:-s, :]], axis=0)
    a = a + shifted
    s *= 2
  return a


def _router_body(x_ref, wr_ref, p01_ref, w0_ref, w1_ref, sp_ref):
  x = x_ref[...]
  wr = wr_ref[...]
  logits = lax.dot_general(x, wr, (((1,), (1,)), ((), ())),
                           preferred_element_type=jnp.float32)
  ii = lax.broadcasted_iota(jnp.int32, (N_TOK, E), 1)
  m1 = jnp.max(logits, axis=1, keepdims=True)
  a1 = jnp.min(jnp.where(logits == m1, ii, E), axis=1, keepdims=True)
  masked = jnp.where(ii == a1, jnp.float32(-jnp.inf), logits)
  m2 = jnp.max(masked, axis=1, keepdims=True)
  a2 = jnp.min(jnp.where(masked == m2, ii, E), axis=1, keepdims=True)
  t = jnp.exp(m2 - m1)
  ones16 = jnp.ones((1, 16), jnp.float32)
  w0_ref[...] = (1.0 / (1.0 + t)) * ones16
  w1_ref[...] = (t / (1.0 + t)) * ones16

  oh0 = (ii == a1).astype(jnp.float32)
  oh1 = (ii == a2).astype(jnp.float32)
  cs01 = _cumsum_rows(jnp.concatenate([oh0, oh1], axis=1))
  cs0 = cs01[:, :E]
  cs1 = cs01[:, E:]
  c0 = cs0[N_TOK - 1:N_TOK, :]
  c1 = cs1[N_TOK - 1:N_TOK, :]
  counts = c0 + c1
  padded = jnp.ceil(counts / BLK) * BLK
  tri = (lax.broadcasted_iota(jnp.int32, (E, E), 0)
         < lax.broadcasted_iota(jnp.int32, (E, E), 1)).astype(jnp.float32)
  off = lax.dot_general(padded, tri, (((1,), (0,)), ((), ())),
                        preferred_element_type=jnp.float32)
  p0 = jnp.sum(oh0 * (off + cs0 - oh0), axis=1, keepdims=True)
  p1 = jnp.sum(oh1 * (off + c0 + cs1 - oh1), axis=1, keepdims=True)
  p01_ref[...] = jnp.concatenate([p0, p1], axis=0).astype(jnp.int32)
  bstart = lax.broadcasted_iota(jnp.int32, (N_BLOCKS, E), 0) * BLK
  offi = off.astype(jnp.int32)
  be = (jnp.sum((bstart >= offi).astype(jnp.int32), axis=1, keepdims=True)
        - 1)
  total = jnp.sum(padded, axis=1, keepdims=True).astype(jnp.int32)
  valid = (bstart[:, :1] < total).astype(jnp.int32)
  ei = lax.broadcasted_iota(jnp.int32, (E, E), 0)
  fi = lax.broadcasted_iota(jnp.int32, (E, E), 1)
  last_e = jnp.max(jnp.where(padded > 0.5,
                             lax.broadcasted_iota(jnp.int32, (1, E), 1), -1),
                   axis=1, keepdims=True)
  be = jnp.where(valid == 1, be, last_e)
  be_prev = jnp.concatenate(
      [jnp.full((1, 1), -1, jnp.int32), be[:-1, :]], axis=0)
  first = (be != be_prev).astype(jnp.int32)
  run = _cumsum_rows(first.astype(jnp.float32)).astype(jnp.int32)
  slot = lax.rem(run - 1, 2)
  cand = jnp.where((fi > ei) & (padded > 0.5), fi, E)
  nxt_e = jnp.min(cand, axis=1, keepdims=True)
  ohb = (lax.broadcasted_iota(jnp.int32, (N_BLOCKS, E), 1) == be)
  nxt = jnp.sum(jnp.where(ohb, jnp.reshape(nxt_e, (1, E)), 0), axis=1,
                keepdims=True)
  nxt = jnp.where(nxt >= E, be, nxt)
  zpad = jnp.zeros((N_BLOCKS, 3), jnp.int32)
  sp_ref[...] = jnp.concatenate([be, first, nxt, valid, slot, zpad], axis=1)


def _router(x_flat, Wr, interpret=False):
  return pl.pallas_call(
      _router_body,
      out_shape=[
          jax.ShapeDtypeStruct((2 * N_TOK, 1), jnp.int32),
          jax.ShapeDtypeStruct((N_TOK, 16), jnp.float32),
          jax.ShapeDtypeStruct((N_TOK, 16), jnp.float32),
          jax.ShapeDtypeStruct((N_BLOCKS, 8), jnp.int32),
      ],
      interpret=interpret,
  )(x_flat, Wr)


def _sc_dispatch_body(x_hbm, p01_hbm, xs_hbm, idx_v, x_v, sem):
  wid = lax.axis_index("s") * NC + lax.axis_index("c")
  base = wid * TPW
  pltpu.sync_copy(x_hbm.at[pl.ds(base, TPW)], x_v)
  pltpu.sync_copy(p01_hbm.at[pl.ds(base, TPW)], idx_v)
  pltpu.async_copy(x_v, xs_hbm.at[idx_v], sem).wait()
  pltpu.sync_copy(p01_hbm.at[pl.ds(N_TOK + base, TPW)], idx_v)
  pltpu.async_copy(x_v, xs_hbm.at[idx_v], sem).wait()


@functools.lru_cache(maxsize=1)
def _sc_dispatch():
  return pl.kernel(
      _sc_dispatch_body,
      out_type=jax.ShapeDtypeStruct((G_ROWS, C), jnp.float32),
      mesh=_sc_mesh(),
      scratch_types=[
          pltpu.VMEM((TPW,), jnp.int32),
          pltpu.VMEM((TPW, C), jnp.float32),
          pltpu.SemaphoreType.DMA,
      ],
  )


def _sc_combine_body(ys_hbm, p01_hbm, sh_hbm, w0_hbm, w1_hbm,
                     out_hbm, idx_v, rows_v, acc_v, wv, sem):
  wid = lax.axis_index("s") * NC + lax.axis_index("c")
  base = wid * TPW
  pltpu.sync_copy(sh_hbm.at[pl.ds(base, TPW)], acc_v)

  def accumulate(p_off, w_hbm):
    pltpu.sync_copy(p01_hbm.at[pl.ds(p_off + base, TPW)], idx_v)
    pltpu.async_copy(ys_hbm.at[idx_v], rows_v, sem).wait()
    pltpu.sync_copy(w_hbm.at[pl.ds(base, TPW)], wv)

    def tok_body(tok, carry):
      wvec = wv[tok]
      for g in range(C // 16):
        sl = pl.ds(g * 16, 16)
        acc_v[tok, sl] = acc_v[tok, sl] + wvec * rows_v[tok, sl]
      return carry

    lax.fori_loop(0, TPW, tok_body, 0)

  accumulate(0, w0_hbm)
  accumulate(N_TOK, w1_hbm)
  pltpu.sync_copy(acc_v, out_hbm.at[pl.ds(base, TPW)])


@functools.lru_cache(maxsize=1)
def _sc_combine():
  return pl.kernel(
      _sc_combine_body,
      out_type=jax.ShapeDtypeStruct((N_TOK, C), jnp.float32),
      mesh=_sc_mesh(),
      scratch_types=[
          pltpu.VMEM((TPW,), jnp.int32),
          pltpu.VMEM((TPW, C), jnp.float32),
          pltpu.VMEM((TPW, C), jnp.float32),
          pltpu.VMEM((TPW, 16), jnp.float32),
          pltpu.SemaphoreType.DMA,
      ],
  )


def _gmm_body(sp_ref, xs_ref, wg_hbm, wu_hbm, wd_hbm, ys_ref,
              wg2, wu2, wd2, sems):
  b = pl.program_id(0)
  cur = sp_ref[b, 0]
  first = sp_ref[b, 1]
  nxt = sp_ref[b, 2]
  valid = sp_ref[b, 3]
  slot = sp_ref[b, 4]

  def wcopies(e, s):
    return [
        pltpu.make_async_copy(wg_hbm.at[e], wg2.at[s], sems.at[s, 0]),
        pltpu.make_async_copy(wu_hbm.at[e], wu2.at[s], sems.at[s, 1]),
        pltpu.make_async_copy(wd_hbm.at[e], wd2.at[s], sems.at[s, 2]),
    ]

  @pl.when(b == 0)
  def _():
    for cp in wcopies(cur, slot):
      cp.start()

  @pl.when(first == 1)
  def _():
    for cp in wcopies(cur, slot):
      cp.wait()

    @pl.when(nxt != cur)
    def _():
      for cp in wcopies(nxt, 1 - slot):
        cp.start()

  @pl.when(valid == 1)
  def _():
    x = xs_ref[...]
    hg = jnp.dot(x, wg2[slot], preferred_element_type=jnp.float32)
    hu = jnp.dot(x, wu2[slot], preferred_element_type=jnp.float32)
    act = hg * lax.logistic(hg) * hu
    ys_ref[...] = jnp.dot(act, wd2[slot], preferred_element_type=jnp.float32)


def _gmm(sp, xs, Wg, Wu, Wd, interpret=False):
  grid_spec = pltpu.PrefetchScalarGridSpec(
      num_scalar_prefetch=1,
      grid=(N_BLOCKS,),
      in_specs=[
          pl.BlockSpec((BLK, C), lambda b, sp: (b, 0)),
          pl.BlockSpec(memory_space=pl.ANY),
          pl.BlockSpec(memory_space=pl.ANY),
          pl.BlockSpec(memory_space=pl.ANY),
      ],
      out_specs=pl.BlockSpec((BLK, C), lambda b, sp: (b, 0)),
      scratch_shapes=[
          pltpu.VMEM((2, C, D_FF), jnp.float32),
          pltpu.VMEM((2, C, D_FF), jnp.float32),
          pltpu.VMEM((2, D_FF, C), jnp.float32),
          pltpu.SemaphoreType.DMA((2, 3)),
      ],
  )
  return pl.pallas_call(
      _gmm_body,
      grid_spec=grid_spec,
      out_shape=jax.ShapeDtypeStruct((G_ROWS, C), jnp.float32),
      compiler_params=pltpu.CompilerParams(
          dimension_semantics=("arbitrary",)),
      interpret=interpret,
  )(sp, xs, Wg, Wu, Wd)


def _shared_body(x_ref, sg_hbm, su_hbm, sd_hbm, out_ref, sgb, sub, sdb, sems):
  b = pl.program_id(0)

  def copies():
    return [
        pltpu.make_async_copy(sg_hbm, sgb, sems.at[0]),
        pltpu.make_async_copy(su_hbm, sub, sems.at[1]),
        pltpu.make_async_copy(sd_hbm, sdb, sems.at[2]),
    ]

  @pl.when(b == 0)
  def _():
    for cp in copies():
      cp.start()

  x = x_ref[...]
  cps = copies()

  @pl.when(b == 0)
  def _():
    cps[0].wait()
  hg = jnp.dot(x, sgb[...], preferred_element_type=jnp.float32)

  @pl.when(b == 0)
  def _():
    cps[1].wait()
  hu = jnp.dot(x, sub[...], preferred_element_type=jnp.float32)
  act = hg * lax.logistic(hg) * hu

  @pl.when(b == 0)
  def _():
    cps[2].wait()
  out_ref[...] = jnp.dot(act, sdb[...], preferred_element_type=jnp.float32)


def _shared(x_flat, Sg, Su, Sd, interpret=False):
  return pl.pallas_call(
      _shared_body,
      grid=(N_TOK // BLK,),
      in_specs=[
          pl.BlockSpec((BLK, C), lambda b: (b, 0)),
          pl.BlockSpec(memory_space=pl.ANY),
          pl.BlockSpec(memory_space=pl.ANY),
          pl.BlockSpec(memory_space=pl.ANY),
      ],
      out_specs=pl.BlockSpec((BLK, C), lambda b: (b, 0)),
      out_shape=jax.ShapeDtypeStruct((N_TOK, C), jnp.float32),
      scratch_shapes=[
          pltpu.VMEM((C, D_FF), jnp.float32),
          pltpu.VMEM((C, D_FF), jnp.float32),
          pltpu.VMEM((D_FF, C), jnp.float32),
          pltpu.SemaphoreType.DMA((3,)),
      ],
      compiler_params=pltpu.CompilerParams(
          dimension_semantics=("arbitrary",)),
      interpret=interpret,
  )(x_flat, Sg, Su, Sd)


def kernel(x, Wr, Wg, Wu, Wd, Sg, Su, Sd):
  x_flat = x.reshape(N_TOK, C)
  p01c, w0b, w1b, sp = _router(x_flat, Wr)
  p01 = p01c.reshape(2 * N_TOK)
  xs = _sc_dispatch()(x_flat, p01)
  sh = _shared(x_flat, Sg, Su, Sd)
  ys = _gmm(sp, xs, Wg, Wu, Wd)
  out = _sc_combine()(ys, p01, sh, w0b, w1b)
  return out.reshape(x.shape)


# speedup vs baseline: 1.0947x; 1.0947x over previous
import functools

import jax
import jax.numpy as jnp
from jax import lax
from jax.experimental import pallas as pl
from jax.experimental.pallas import tpu as pltpu
from jax.experimental.pallas import tpu_sc as plsc

N_TOK = 2048
C = 768
E = 8
D_FF = 2048
BLK = 128
G_ROWS = 2 * N_TOK + E * BLK
N_BLOCKS = G_ROWS // BLK
NC, NS = 2, 16
NW = NC * NS
TPW = N_TOK // NW
CHK = 32
N_CHK = TPW // CHK


@functools.lru_cache(maxsize=1)
def _sc_mesh():
  return plsc.VectorSubcoreMesh(
      core_axis_name="c", subcore_axis_name="s", num_cores=NC, num_subcores=NS)


def _cumsum_rows(a):
  n = a.shape[0]
  s = 1
  while s < n:
    shifted = jnp.concatenate(
        [jnp.zeros((s, a.shape[1]), a.dtype), a[:-s, :]], axis=0)
    a = a + shifted
    s *= 2
  return a


def _router_body(x_ref, wr_ref, p01_ref, w0_ref, w1_ref, sp_ref):
  x = x_ref[...]
  wr = wr_ref[...]
  logits = lax.dot_general(x, wr, (((1,), (1,)), ((), ())),
                           preferred_element_type=jnp.float32)
  ii = lax.broadcasted_iota(jnp.int32, (N_TOK, E), 1)
  m1 = jnp.max(logits, axis=1, keepdims=True)
  a1 = jnp.min(jnp.where(logits == m1, ii, E), axis=1, keepdims=True)
  masked = jnp.where(ii == a1, jnp.float32(-jnp.inf), logits)
  m2 = jnp.max(masked, axis=1, keepdims=True)
  a2 = jnp.min(jnp.where(masked == m2, ii, E), axis=1, keepdims=True)
  t = jnp.exp(m2 - m1)
  w0_ref[...] = 1.0 / (1.0 + t)
  w1_ref[...] = t / (1.0 + t)

  oh0 = (ii == a1).astype(jnp.float32)
  oh1 = (ii == a2).astype(jnp.float32)
  cs01 = _cumsum_rows(jnp.concatenate([oh0, oh1], axis=1))
  cs0 = cs01[:, :E]
  cs1 = cs01[:, E:]
  c0 = cs0[N_TOK - 1:N_TOK, :]
  c1 = cs1[N_TOK - 1:N_TOK, :]
  counts = c0 + c1
  padded = jnp.ceil(counts / BLK) * BLK
  tri = (lax.broadcasted_iota(jnp.int32, (E, E), 0)
         < lax.broadcasted_iota(jnp.int32, (E, E), 1)).astype(jnp.float32)
  off = lax.dot_general(padded, tri, (((1,), (0,)), ((), ())),
                        preferred_element_type=jnp.float32)
  p0 = jnp.sum(oh0 * (off + cs0 - oh0), axis=1, keepdims=True)
  p1 = jnp.sum(oh1 * (off + c0 + cs1 - oh1), axis=1, keepdims=True)
  p01_ref[...] = jnp.concatenate([p0, p1], axis=0).astype(jnp.int32)
  bstart = lax.broadcasted_iota(jnp.int32, (N_BLOCKS, E), 0) * BLK
  offi = off.astype(jnp.int32)
  be = (jnp.sum((bstart >= offi).astype(jnp.int32), axis=1, keepdims=True)
        - 1)
  total = jnp.sum(padded, axis=1, keepdims=True).astype(jnp.int32)
  valid = (bstart[:, :1] < total).astype(jnp.int32)
  ei = lax.broadcasted_iota(jnp.int32, (E, E), 0)
  fi = lax.broadcasted_iota(jnp.int32, (E, E), 1)
  last_e = jnp.max(jnp.where(padded > 0.5,
                             lax.broadcasted_iota(jnp.int32, (1, E), 1), -1),
                   axis=1, keepdims=True)
  be = jnp.where(valid == 1, be, last_e)
  be_prev = jnp.concatenate(
      [jnp.full((1, 1), -1, jnp.int32), be[:-1, :]], axis=0)
  first = (be != be_prev).astype(jnp.int32)
  run = _cumsum_rows(first.astype(jnp.float32)).astype(jnp.int32)
  slot = lax.rem(run - 1, 2)
  cand = jnp.where((fi > ei) & (padded > 0.5), fi, E)
  nxt_e = jnp.min(cand, axis=1, keepdims=True)
  ohb = (lax.broadcasted_iota(jnp.int32, (N_BLOCKS, E), 1) == be)
  nxt = jnp.sum(jnp.where(ohb, jnp.reshape(nxt_e, (1, E)), 0), axis=1,
                keepdims=True)
  nxt = jnp.where(nxt >= E, be, nxt)
  zpad = jnp.zeros((N_BLOCKS, 3), jnp.int32)
  sp_ref[...] = jnp.concatenate([be, first, nxt, valid, slot, zpad], axis=1)


def _router(x_flat, Wr, interpret=False):
  return pl.pallas_call(
      _router_body,
      out_shape=[
          jax.ShapeDtypeStruct((2 * N_TOK, 1), jnp.int32),
          jax.ShapeDtypeStruct((N_TOK, 1), jnp.float32),
          jax.ShapeDtypeStruct((N_TOK, 1), jnp.float32),
          jax.ShapeDtypeStruct((N_BLOCKS, 8), jnp.int32),
      ],
      interpret=interpret,
  )(x_flat, Wr)


def _sc_dispatch_body(x_hbm, p01_hbm, xs_hbm, idx_v, x_v, sem):
  wid = lax.axis_index("s") * NC + lax.axis_index("c")
  base = wid * TPW
  pltpu.sync_copy(x_hbm.at[pl.ds(base, TPW)], x_v)
  pltpu.sync_copy(p01_hbm.at[pl.ds(base, TPW)], idx_v)
  pltpu.async_copy(x_v, xs_hbm.at[idx_v], sem).wait()
  pltpu.sync_copy(p01_hbm.at[pl.ds(N_TOK + base, TPW)], idx_v)
  pltpu.async_copy(x_v, xs_hbm.at[idx_v], sem).wait()


@functools.lru_cache(maxsize=1)
def _sc_dispatch():
  return pl.kernel(
      _sc_dispatch_body,
      out_type=jax.ShapeDtypeStruct((G_ROWS, C), jnp.float32),
      mesh=_sc_mesh(),
      scratch_types=[
          pltpu.VMEM((TPW,), jnp.int32),
          pltpu.VMEM((TPW, C), jnp.float32),
          pltpu.SemaphoreType.DMA,
      ],
  )


def _sc_combine_body(ys_hbm, p01_hbm, z0_hbm, z1_hbm, idx_v, rows_v, sem):
  wid = lax.axis_index("s") * NC + lax.axis_index("c")
  base = wid * TPW
  pltpu.sync_copy(p01_hbm.at[pl.ds(base, TPW)], idx_v)
  pltpu.async_copy(ys_hbm.at[idx_v], rows_v, sem).wait()
  pltpu.sync_copy(rows_v, z0_hbm.at[pl.ds(base, TPW)])
  pltpu.sync_copy(p01_hbm.at[pl.ds(N_TOK + base, TPW)], idx_v)
  pltpu.async_copy(ys_hbm.at[idx_v], rows_v, sem).wait()
  pltpu.sync_copy(rows_v, z1_hbm.at[pl.ds(base, TPW)])


@functools.lru_cache(maxsize=1)
def _sc_combine():
  return pl.kernel(
      _sc_combine_body,
      out_type=(jax.ShapeDtypeStruct((N_TOK, C), jnp.float32),
                jax.ShapeDtypeStruct((N_TOK, C), jnp.float32)),
      mesh=_sc_mesh(),
      scratch_types=[
          pltpu.VMEM((TPW,), jnp.int32),
          pltpu.VMEM((TPW, C), jnp.float32),
          pltpu.SemaphoreType.DMA,
      ],
  )


def _gmm_body(sp_ref, xs_ref, wg_hbm, wu_hbm, wd_hbm, ys_ref,
              wg2, wu2, wd2, sems):
  b = pl.program_id(0)
  cur = sp_ref[b, 0]
  first = sp_ref[b, 1]
  nxt = sp_ref[b, 2]
  valid = sp_ref[b, 3]
  slot = sp_ref[b, 4]

  def wcopies(e, s):
    return [
        pltpu.make_async_copy(wg_hbm.at[e], wg2.at[s], sems.at[s, 0]),
        pltpu.make_async_copy(wu_hbm.at[e], wu2.at[s], sems.at[s, 1]),
        pltpu.make_async_copy(wd_hbm.at[e], wd2.at[s], sems.at[s, 2]),
    ]

  @pl.when(b == 0)
  def _():
    for cp in wcopies(cur, slot):
      cp.start()

  @pl.when(first == 1)
  def _():
    for cp in wcopies(cur, slot):
      cp.wait()

    @pl.when(nxt != cur)
    def _():
      for cp in wcopies(nxt, 1 - slot):
        cp.start()

  @pl.when(valid == 1)
  def _():
    x = xs_ref[...]
    hg = jnp.dot(x, wg2[slot], preferred_element_type=jnp.float32)
    hu = jnp.dot(x, wu2[slot], preferred_element_type=jnp.float32)
    act = hg * lax.logistic(hg) * hu
    ys_ref[...] = jnp.dot(act, wd2[slot], preferred_element_type=jnp.float32)


def _gmm(sp, xs, Wg, Wu, Wd, interpret=False):
  grid_spec = pltpu.PrefetchScalarGridSpec(
      num_scalar_prefetch=1,
      grid=(N_BLOCKS,),
      in_specs=[
          pl.BlockSpec((BLK, C), lambda b, sp: (b, 0)),
          pl.BlockSpec(memory_space=pl.ANY),
          pl.BlockSpec(memory_space=pl.ANY),
          pl.BlockSpec(memory_space=pl.ANY),
      ],
      out_specs=pl.BlockSpec((BLK, C), lambda b, sp: (b, 0)),
      scratch_shapes=[
          pltpu.VMEM((2, C, D_FF), jnp.float32),
          pltpu.VMEM((2, C, D_FF), jnp.float32),
          pltpu.VMEM((2, D_FF, C), jnp.float32),
          pltpu.SemaphoreType.DMA((2, 3)),
      ],
  )
  return pl.pallas_call(
      _gmm_body,
      grid_spec=grid_spec,
      out_shape=jax.ShapeDtypeStruct((G_ROWS, C), jnp.float32),
      compiler_params=pltpu.CompilerParams(
          dimension_semantics=("arbitrary",)),
      interpret=interpret,
  )(sp, xs, Wg, Wu, Wd)


def _shared_body(x_ref, sg_ref, su_ref, sd_ref, z0_ref, z1_ref,
                 w0_ref, w1_ref, out_ref):
  x = x_ref[...]
  hg = jnp.dot(x, sg_ref[...], preferred_element_type=jnp.float32)
  hu = jnp.dot(x, su_ref[...], preferred_element_type=jnp.float32)
  act = hg * lax.logistic(hg) * hu
  part = jnp.dot(act, sd_ref[...], preferred_element_type=jnp.float32)
  out_ref[...] = part + w0_ref[...] * z0_ref[...] + w1_ref[...] * z1_ref[...]


def _shared(x_flat, Sg, Su, Sd, z0, z1, w0, w1, interpret=False):
  return pl.pallas_call(
      _shared_body,
      grid=(N_TOK // BLK,),
      in_specs=[
          pl.BlockSpec((BLK, C), lambda b: (b, 0)),
          pl.BlockSpec((C, D_FF), lambda b: (0, 0)),
          pl.BlockSpec((C, D_FF), lambda b: (0, 0)),
          pl.BlockSpec((D_FF, C), lambda b: (0, 0)),
          pl.BlockSpec((BLK, C), lambda b: (b, 0)),
          pl.BlockSpec((BLK, C), lambda b: (b, 0)),
          pl.BlockSpec((BLK, 1), lambda b: (b, 0)),
          pl.BlockSpec((BLK, 1), lambda b: (b, 0)),
      ],
      out_specs=pl.BlockSpec((BLK, C), lambda b: (b, 0)),
      out_shape=jax.ShapeDtypeStruct((N_TOK, C), jnp.float32),
      compiler_params=pltpu.CompilerParams(
          dimension_semantics=("arbitrary",)),
      interpret=interpret,
  )(x_flat, Sg, Su, Sd, z0, z1, w0, w1)


def kernel(x, Wr, Wg, Wu, Wd, Sg, Su, Sd):
  x_flat = x.reshape(N_TOK, C)
  p01c, w0, w1, sp = _router(x_flat, Wr)
  p01 = p01c.reshape(2 * N_TOK)
  xs = _sc_dispatch()(x_flat, p01)
  ys = _gmm(sp, xs, Wg, Wu, Wd)
  z0, z1 = _sc_combine()(ys, p01)
  out = _shared(x_flat, Sg, Su, Sd, z0, z1, w0, w1)
  return out.reshape(x.shape)
